# Initial kernel scaffold; baseline (speedup 1.0000x reference)
import jax
import jax.numpy as jnp
from jax.experimental import pallas as pl


def kernel(brand, material, author, color, price, W_price, bn_gamma, bn_beta, brand_table, material_table, author_table, color_table):
    raise NotImplementedError("write your pallas kernel here")


# speedup vs baseline: 6.4225x; 6.4225x over previous
import functools

import jax
import jax.numpy as jnp
from jax import lax
from jax.experimental import pallas as pl
from jax.experimental.pallas import tpu as pltpu
from jax.experimental.pallas import tpu_sc as plsc

B, L, D = 4096, 50, 64
V = 100000
EPS = 1e-5
N = B * L
NC, NS = 2, 16
NW = NC * NS
C = 128
ROWS_W = N // NW
NCH = ROWS_W // C
NG = N // C


def _stats_body(p_ref, w_ref, g_ref, b_ref, out_ref):
    p = p_ref[...]
    s1 = jnp.sum(p)
    s2 = jnp.sum(p * p)
    mean = s1 / N
    var = s2 / N - mean * mean
    w = w_ref[...]
    scale = w * g_ref[...] * lax.rsqrt(w * w * var + EPS)
    off = b_ref[...] - mean * scale
    out_ref[...] = jnp.concatenate([scale, off], axis=0)


def _price_affine(price2d, w, gamma, beta):
    return pl.pallas_call(
        _stats_body,
        out_shape=jax.ShapeDtypeStruct((2, D), jnp.float32),
    )(price2d, w, gamma, beta)


def _sc_body(brand, material, author, color, price, so_tbl,
             t_brand, t_material, t_author, t_color, out,
             idx_v, x_v, so_v, g_v, o_v, sem):
    wid = lax.axis_index("s") * NC + lax.axis_index("c")
    r0 = wid * ROWS_W

    pltpu.sync_copy(brand.at[pl.ds(r0, ROWS_W)], idx_v.at[0])
    pltpu.sync_copy(material.at[pl.ds(r0, ROWS_W)], idx_v.at[1])
    pltpu.sync_copy(author.at[pl.ds(r0, ROWS_W)], idx_v.at[2])
    pltpu.sync_copy(color.at[pl.ds(r0, ROWS_W)], idx_v.at[3])
    pltpu.sync_copy(price.at[pl.ds(r0, ROWS_W)], x_v)
    pltpu.sync_copy(so_tbl, so_v)

    def chunk(c, _):
        cb = c * C
        cps = [pltpu.async_copy(t.at[idx_v.at[f, pl.ds(cb, C)]], g_v.at[f], sem)
               for f, t in enumerate((t_brand, t_material, t_author, t_color))]
        for cp in cps:
            cp.wait()

        def row(j, _):
            j16 = jnp.full((16,), cb + j, jnp.int32)
            xs = plsc.load_gather(x_v, [j16])
            ms = []
            for f in range(4):
                f16 = jnp.full((16,), f, jnp.int32)
                iv = plsc.load_gather(idx_v, [f16, j16])
                ms.append(jnp.where(iv != 0, 1.0, 0.0).astype(jnp.float32))
            for blk in range(4):
                dsl = pl.ds(blk * 16, 16)
                acc = (ms[0] * g_v[0, j, dsl] + ms[1] * g_v[1, j, dsl]
                       + ms[2] * g_v[2, j, dsl] + ms[3] * g_v[3, j, dsl])
                pr = jnp.maximum(xs * so_v[0, dsl] + so_v[1, dsl], 0.0)
                o_v[j, dsl] = acc + pr
            return _

        lax.fori_loop(0, C, row, None)
        pltpu.sync_copy(o_v, out.at[pl.ds(r0 + cb, C)])
        return _

    lax.fori_loop(0, NCH, chunk, None)


_sc_call = functools.partial(
    pl.kernel,
    out_type=jax.ShapeDtypeStruct((N, D), jnp.float32),
    mesh=plsc.VectorSubcoreMesh(core_axis_name="c", subcore_axis_name="s"),
    compiler_params=pltpu.CompilerParams(
        needs_layout_passes=False, use_tc_tiling_on_sc=False),
    scratch_types=[
        pltpu.VMEM((4, ROWS_W), jnp.int32),
        pltpu.VMEM((ROWS_W,), jnp.float32),
        pltpu.VMEM((2, D), jnp.float32),
        pltpu.VMEM((4, C, D), jnp.float32),
        pltpu.VMEM((C, D), jnp.float32),
        pltpu.SemaphoreType.DMA,
    ],
)


def kernel(brand, material, author, color, price, W_price, bn_gamma, bn_beta,
           brand_table, material_table, author_table, color_table):
    so_tbl = _price_affine(price.reshape(NG, C), W_price,
                           bn_gamma.reshape(1, D), bn_beta.reshape(1, D))
    sc = _sc_call(_sc_body)
    out = sc(brand.reshape(N), material.reshape(N),
             author.reshape(N), color.reshape(N),
             price.reshape(N), so_tbl,
             brand_table, material_table, author_table, color_table)
    return out.reshape(B, L, D)
